# Initial kernel scaffold; baseline (speedup 1.0000x reference)
import jax
import jax.numpy as jnp
from jax.experimental import pallas as pl


def kernel(x, W, bias, input_scope, is_train):
    raise NotImplementedError("write your pallas kernel here")


# speedup vs baseline: 3.1526x; 3.1526x over previous
import functools

import jax
import jax.numpy as jnp
from jax import lax
from jax.experimental import pallas as pl
from jax.experimental.pallas import tpu as pltpu
from jax.experimental.pallas import tpu_sc as plsc

NC = 2
NS = 16
LANES = 16

CHUNK = 128


def _sc_partial_sums(x, cuts, total, d, nbags):
    nworkers = NC * NS
    rows_per_worker = total // nworkers
    nchunks = rows_per_worker // CHUNK
    nj = d // LANES
    mesh = plsc.VectorSubcoreMesh(core_axis_name="c", subcore_axis_name="s")

    @functools.partial(
        pl.kernel,
        out_type=jax.ShapeDtypeStruct((nworkers, nbags, d), jnp.float32),
        mesh=mesh,
        scratch_types=[
            pltpu.VMEM((2, CHUNK, d), jnp.float32),
            pltpu.VMEM((nbags,), jnp.int32),
            pltpu.VMEM((nbags, d), jnp.float32),
            pltpu.SemaphoreType.DMA,
            pltpu.SemaphoreType.DMA,
        ],
    )
    def body(x_hbm, cuts_hbm, out_hbm, xbuf, cutsv, acc, sem0, sem1):
        c = lax.axis_index("c")
        s = lax.axis_index("s")
        wid = c * NS + s
        base = wid * rows_per_worker

        pltpu.sync_copy(cuts_hbm, cutsv)

        def zrow(i, carry):
            for j in range(nj):
                acc[i, pl.ds(j * LANES, LANES)] = jnp.zeros(
                    (LANES,), jnp.float32)
            return carry
        lax.fori_loop(0, nbags, zrow, 0)

        sems = (sem0, sem1)
        copies = [
            pltpu.make_async_copy(
                x_hbm.at[pl.ds(base + g * CHUNK, CHUNK)],
                xbuf.at[g % 2],
                sems[g % 2],
            )
            for g in range(nchunks)
        ]
        copies[0].start()

        cuts_vec = cutsv[...]
        cut_s = [cuts_vec[i] for i in range(nbags)]

        for g in range(nchunks):
            if g + 1 < nchunks:
                copies[g + 1].start()
            copies[g].wait()
            slot = g % 2
            cs = base + g * CHUNK
            b0 = jnp.int32(0)
            for i in range(nbags):
                b0 = b0 + jnp.where(cut_s[i] <= cs, 1, 0).astype(jnp.int32)

            def run_step(_, carry):
                rl, b = carry
                cut_next = jnp.int32(0)
                for i in range(nbags):
                    cut_next = cut_next + jnp.where(
                        b == i, cut_s[i], 0).astype(jnp.int32)
                re = jnp.minimum(cut_next - cs, CHUNK)

                def row(rr, c):
                    for j in range(nj):
                        plsc.addupdate(
                            acc.at[b, pl.ds(j * LANES, LANES)],
                            xbuf[slot, rr, pl.ds(j * LANES, LANES)])
                    return c
                lax.fori_loop(rl, re, row, 0)
                return re, jnp.minimum(b + 1, nbags - 1)

            lax.fori_loop(0, nbags, run_step, (jnp.int32(0), b0))

        pltpu.sync_copy(acc, out_hbm.at[wid])

    return body(x, cuts)


def _tc_project(partials, wt, bias2d, lengths, flag):
    nbags = partials.shape[1]
    ncls = wt.shape[1]

    def body(part_ref, wt_ref, bias_ref, len_ref, flag_ref, out_ref):
        sums = jnp.sum(part_ref[...], axis=0)
        means = sums / len_ref[...]
        logits = jnp.dot(means, wt_ref[...],
                         preferred_element_type=jnp.float32) + bias_ref[...]
        mx = jnp.max(logits, axis=1, keepdims=True)
        e = jnp.exp(logits - mx)
        sm = e / jnp.sum(e, axis=1, keepdims=True)
        out_ref[...] = jnp.where(flag_ref[0, 0] == 1, logits, sm)

    return pl.pallas_call(
        body,
        out_shape=jax.ShapeDtypeStruct((nbags, ncls), jnp.float32),
        in_specs=[
            pl.BlockSpec(memory_space=pltpu.VMEM),
            pl.BlockSpec(memory_space=pltpu.VMEM),
            pl.BlockSpec(memory_space=pltpu.VMEM),
            pl.BlockSpec(memory_space=pltpu.VMEM),
            pl.BlockSpec(memory_space=pltpu.SMEM),
        ],
    )(partials, wt, bias2d, lengths, flag)


def kernel(x, W, bias, input_scope, is_train):
    total, d = x.shape
    scope = jnp.asarray(input_scope, jnp.int32)
    nbags = scope.shape[0] - 1
    cuts = scope[1:]
    lengths = (scope[1:] - scope[:-1]).astype(jnp.float32).reshape(nbags, 1)
    flag = jnp.asarray(is_train, jnp.int32).reshape(1, 1)

    partials = _sc_partial_sums(x, cuts, total, d, nbags)
    return _tc_project(partials, W.T, bias.reshape(1, -1), lengths, flag)


# speedup vs baseline: 5.7006x; 1.8082x over previous
import functools

import jax
import jax.numpy as jnp
from jax import lax
from jax.experimental import pallas as pl
from jax.experimental.pallas import tpu as pltpu
from jax.experimental.pallas import tpu_sc as plsc

NC = 2
NS = 16
LANES = 16

CHUNK = 128


def _sc_partial_sums(x, cuts, total, d, nbags):
    nworkers = NC * NS
    rows_per_worker = total // nworkers
    nchunks = rows_per_worker // CHUNK
    nj = d // LANES
    mesh = plsc.VectorSubcoreMesh(core_axis_name="c", subcore_axis_name="s")

    @functools.partial(
        pl.kernel,
        out_type=jax.ShapeDtypeStruct((nworkers, nbags, d), jnp.float32),
        mesh=mesh,
        scratch_types=[
            pltpu.VMEM((2, CHUNK, d), jnp.float32),
            pltpu.VMEM((nbags,), jnp.int32),
            pltpu.VMEM((nbags, d), jnp.float32),
            pltpu.SemaphoreType.DMA,
            pltpu.SemaphoreType.DMA,
        ],
    )
    def body(x_hbm, cuts_hbm, out_hbm, xbuf, cutsv, acc, sem0, sem1):
        c = lax.axis_index("c")
        s = lax.axis_index("s")
        wid = c * NS + s
        base = wid * rows_per_worker

        pltpu.sync_copy(cuts_hbm, cutsv)

        def zrow(i, carry):
            for j in range(nj):
                acc[i, pl.ds(j * LANES, LANES)] = jnp.zeros(
                    (LANES,), jnp.float32)
            return carry
        lax.fori_loop(0, nbags, zrow, 0)

        pltpu.make_async_copy(
            x_hbm.at[pl.ds(base, CHUNK)], xbuf.at[0], sem0).start()

        cuts_vec = cutsv[...]
        cut_s = [cuts_vec[i] for i in range(nbags)]

        def bag_of(row):
            b = jnp.int32(0)
            for i in range(nbags):
                b = b + jnp.where(cut_s[i] <= row, 1, 0).astype(jnp.int32)
            return b

        def cut_at(b):
            cn = jnp.int32(0)
            for i in range(nbags):
                cn = cn + jnp.where(b == i, cut_s[i], 0).astype(jnp.int32)
            return cn

        GRP = 16

        def chunk_body(g, carry):
            nxt = g + 1
            even_nxt = lax.rem(nxt, 2) == 0

            @pl.when((nxt < nchunks) & even_nxt)
            def _start_even():
                pltpu.make_async_copy(
                    x_hbm.at[pl.ds(base + nxt * CHUNK, CHUNK)],
                    xbuf.at[0], sem0).start()

            @pl.when((nxt < nchunks) & jnp.logical_not(even_nxt))
            def _start_odd():
                pltpu.make_async_copy(
                    x_hbm.at[pl.ds(base + nxt * CHUNK, CHUNK)],
                    xbuf.at[1], sem1).start()

            even_cur = lax.rem(g, 2) == 0

            @pl.when(even_cur)
            def _wait_even():
                pltpu.make_async_copy(
                    x_hbm.at[pl.ds(base, CHUNK)], xbuf.at[0], sem0).wait()

            @pl.when(jnp.logical_not(even_cur))
            def _wait_odd():
                pltpu.make_async_copy(
                    x_hbm.at[pl.ds(base, CHUNK)], xbuf.at[1], sem1).wait()

            slot = lax.rem(g, 2)
            cs = base + g * CHUNK

            def group(v, c):
                r0 = v * GRP
                row0 = cs + r0
                b = bag_of(row0)
                cutn = cut_at(b)
                uniform = cutn >= row0 + GRP

                def fast():
                    for j in range(nj):
                        sl = pl.ds(j * LANES, LANES)
                        vals = [xbuf[slot, r0 + k, sl] for k in range(GRP)]
                        while len(vals) > 1:
                            vals = [
                                vals[2 * i] + vals[2 * i + 1]
                                for i in range(len(vals) // 2)
                            ] + vals[2 * (len(vals) // 2):]
                        plsc.addupdate(acc.at[b, sl], vals[0])

                def slow():
                    for k in range(GRP):
                        bk = bag_of(row0 + k)
                        for j in range(nj):
                            sl = pl.ds(j * LANES, LANES)
                            plsc.addupdate(acc.at[bk, sl],
                                           xbuf[slot, r0 + k, sl])

                lax.cond(uniform, fast, slow)
                return c

            lax.fori_loop(0, CHUNK // GRP, group, 0)
            return carry

        lax.fori_loop(0, nchunks, chunk_body, 0)

        pltpu.sync_copy(acc, out_hbm.at[wid])

    return body(x, cuts)


def _tc_project(partials, wt, bias2d, lengths, flag):
    nbags = partials.shape[1]
    ncls = wt.shape[1]

    def body(part_ref, wt_ref, bias_ref, len_ref, flag_ref, out_ref):
        sums = jnp.sum(part_ref[...], axis=0)
        means = sums / len_ref[...]
        logits = jnp.dot(means, wt_ref[...],
                         preferred_element_type=jnp.float32) + bias_ref[...]
        mx = jnp.max(logits, axis=1, keepdims=True)
        e = jnp.exp(logits - mx)
        sm = e / jnp.sum(e, axis=1, keepdims=True)
        out_ref[...] = jnp.where(flag_ref[0, 0] == 1, logits, sm)

    return pl.pallas_call(
        body,
        out_shape=jax.ShapeDtypeStruct((nbags, ncls), jnp.float32),
        in_specs=[
            pl.BlockSpec(memory_space=pltpu.VMEM),
            pl.BlockSpec(memory_space=pltpu.VMEM),
            pl.BlockSpec(memory_space=pltpu.VMEM),
            pl.BlockSpec(memory_space=pltpu.VMEM),
            pl.BlockSpec(memory_space=pltpu.SMEM),
        ],
    )(partials, wt, bias2d, lengths, flag)


def kernel(x, W, bias, input_scope, is_train):
    total, d = x.shape
    scope = jnp.asarray(input_scope, jnp.int32)
    nbags = scope.shape[0] - 1
    cuts = scope[1:]
    lengths = (scope[1:] - scope[:-1]).astype(jnp.float32).reshape(nbags, 1)
    flag = jnp.asarray(is_train, jnp.int32).reshape(1, 1)

    partials = _sc_partial_sums(x, cuts, total, d, nbags)
    return _tc_project(partials, W.T, bias.reshape(1, -1), lengths, flag)
